# baseline (speedup 1.0000x reference)
import functools

import jax
import jax.numpy as jnp
from jax import lax
from jax.experimental import pallas as pl
from jax.experimental.pallas import tpu as pltpu
from jax.experimental.pallas import tpu_sc as plsc

B = 16384
V = 1000000
D = 64
P = 64

_NC = 2
_NS = 16
_NW = _NC * _NS
_B_PER_W = B // _NW


@functools.cache
def _make_sc_gather():
    mesh = plsc.VectorSubcoreMesh(core_axis_name="c", subcore_axis_name="s")

    @functools.partial(
        pl.kernel,
        mesh=mesh,
        out_type=jax.ShapeDtypeStruct((B, D), jnp.float32),
        scratch_types=[
            pltpu.VMEM((_B_PER_W,), jnp.int32),
            pltpu.VMEM((_B_PER_W, D), jnp.float32),
            pltpu.SemaphoreType.DMA,
        ],
        compiler_params=pltpu.CompilerParams(use_tc_tiling_on_sc=False),
    )
    def gather_kernel(table_hbm, idx_hbm, out_hbm, idx_v, rows_v, sem):
        wid = lax.axis_index("s") * _NC + lax.axis_index("c")
        base = wid * _B_PER_W
        pltpu.sync_copy(idx_hbm.at[pl.ds(base, _B_PER_W)], idx_v)
        pltpu.async_copy(table_hbm.at[idx_v], rows_v, sem).wait()
        pltpu.sync_copy(rows_v, out_hbm.at[pl.ds(base, _B_PER_W)])

    return gather_kernel


def _dense_body(ue_ref, pf_ref, w1_ref, b1_ref, w2_ref, b2_ref,
                wfu_ref, wfp_ref, bf_ref, out_ref):
    prec = lax.Precision.HIGHEST
    h = jnp.maximum(
        jnp.dot(pf_ref[...], w1_ref[...], precision=prec) + b1_ref[...], 0.0)
    pe = jnp.dot(h, w2_ref[...], precision=prec) + b2_ref[...]
    acc = (jnp.dot(ue_ref[...], wfu_ref[...], precision=prec)
           + jnp.dot(pe, wfp_ref[...], precision=prec)
           + bf_ref[...])
    out_ref[...] = jnp.tanh(acc)


_BM = 1024


def _dense(ue, pf, W1, b1, W2, b2, Wfu, Wfp, bf):
    grid = (B // _BM,)

    def full(r, c):
        return pl.BlockSpec((r, c), lambda i: (0, 0))

    return pl.pallas_call(
        _dense_body,
        grid=grid,
        in_specs=[
            pl.BlockSpec((_BM, D), lambda i: (i, 0)),
            pl.BlockSpec((_BM, P), lambda i: (i, 0)),
            full(P, D // 2),
            full(1, D // 2),
            full(D // 2, D),
            full(1, D),
            full(D, D),
            full(D, D),
            full(1, D),
        ],
        out_specs=pl.BlockSpec((_BM, D), lambda i: (i, 0)),
        out_shape=jax.ShapeDtypeStruct((B, D), jnp.float32),
    )(ue, pf, W1, b1, W2, b2, Wfu, Wfp, bf)


def kernel(user_ids, profile_features, table, W1, b1, W2, b2, Wf, bf):
    user_emb = _make_sc_gather()(table, user_ids.astype(jnp.int32))
    return _dense(
        user_emb, profile_features,
        W1, b1.reshape(1, -1), W2, b2.reshape(1, -1),
        Wf[:D], Wf[D:], bf.reshape(1, -1),
    )
